# baseline (speedup 1.0000x reference)
import jax
import jax.numpy as jnp
from jax.experimental import pallas as pl


def _body(m_ref, x_ref, o_ref):
    x = x_ref[0]
    m = m_ref[0, 0]
    s = jnp.sum(x * m[:, None], axis=0)
    c = jnp.sum(m)
    o_ref[0, 0] = s / c


def kernel(inputs, mask):
    B, T, D = inputs.shape
    m = mask.astype(inputs.dtype).reshape(B, 1, T)
    return pl.pallas_call(
        _body,
        grid=(B,),
        in_specs=[
            pl.BlockSpec((1, 1, T), lambda b: (b, 0, 0)),
            pl.BlockSpec((1, T, D), lambda b: (b, 0, 0)),
        ],
        out_specs=pl.BlockSpec((1, 1, D), lambda b: (b, 0, 0)),
        out_shape=jax.ShapeDtypeStruct((B, 1, D), inputs.dtype),
    )(m, inputs).reshape(B, D)
